# baseline (speedup 1.0000x reference)
import functools

import jax
import jax.numpy as jnp
from jax import lax
from jax.experimental import pallas as pl
from jax.experimental.pallas import tpu as pltpu
from jax.experimental.pallas import tpu_sc as plsc

_EPS = 1e-5

N_GRID = 50000
N_MESH = 10000
E_G2M = 100000
E_M2M = 100000
E_M2G = 150000


def _mlp_body(group_sizes, has_res, use_ln, d_real1, *refs):
    idx = 0
    xs = []
    for g in group_sizes:
        acc = refs[idx][...].astype(jnp.float32)
        for t in range(1, g):
            acc = acc + refs[idx + t][...].astype(jnp.float32)
        idx += g
        xs.append(acc)
    x = xs[0] if len(xs) == 1 else jnp.concatenate(xs, axis=-1)
    res = None
    if has_res:
        res = refs[idx][...]
        idx += 1
    W1 = refs[idx][...]
    b1 = refs[idx + 1][...]
    idx += 2
    if use_ln:
        g1 = refs[idx][...]
        be1 = refs[idx + 1][...]
        idx += 2
    W2 = refs[idx][...]
    b2 = refs[idx + 1][...]
    idx += 2
    if use_ln:
        g2 = refs[idx][...]
        be2 = refs[idx + 1][...]
        idx += 2
    out_ref = refs[idx]

    h = jnp.dot(x.astype(jnp.bfloat16), W1.astype(jnp.bfloat16),
                preferred_element_type=jnp.float32) + b1
    if use_ln:
        H = h.shape[-1]
        if d_real1 == H:
            m = jnp.mean(h, axis=-1, keepdims=True)
            hc = h - m
        else:
            m = jnp.sum(h, axis=-1, keepdims=True) / d_real1
            mask = lax.broadcasted_iota(jnp.int32, h.shape, 1) < d_real1
            hc = jnp.where(mask, h - m, 0.0)
        v = jnp.sum(hc * hc, axis=-1, keepdims=True) / d_real1
        h = hc * lax.rsqrt(v + _EPS) * g1 + be1
    h = h * (1.0 / (1.0 + jnp.exp(-h)))
    y = jnp.dot(h.astype(jnp.bfloat16), W2.astype(jnp.bfloat16),
                preferred_element_type=jnp.float32) + b2
    if use_ln:
        m2 = jnp.mean(y, axis=-1, keepdims=True)
        yc = y - m2
        v2 = jnp.mean(yc * yc, axis=-1, keepdims=True)
        y = yc * lax.rsqrt(v2 + _EPS) * g2 + be2
    if has_res:
        y = y + res
    out_ref[...] = y


def _mlp(parts, p, block_r, *, residual=None, use_ln=True, d_real1=None,
         W1=None):
    N = parts[0][0].shape[0]
    assert N % block_r == 0, (N, block_r)
    W1 = p['W1'] if W1 is None else W1
    W2 = p['W2']
    H = W1.shape[1]
    F = W2.shape[1]
    if d_real1 is None:
        d_real1 = H
    group_sizes = tuple(len(g) for g in parts)
    flat = [a for g in parts for a in g]

    ins = []
    specs = []
    for a in flat:
        ins.append(a)
        specs.append(pl.BlockSpec((block_r, a.shape[1]), lambda i: (i, 0)))
    has_res = residual is not None
    if has_res:
        ins.append(residual)
        specs.append(pl.BlockSpec((block_r, F), lambda i: (i, 0)))

    def add_w(w):
        ins.append(w)
        specs.append(pl.BlockSpec(w.shape, lambda i: (0,) * w.ndim))

    add_w(W1)
    add_w(p['b1'].reshape(1, H))
    if use_ln:
        add_w(p['g1'].reshape(1, H))
        add_w(p['be1'].reshape(1, H))
    add_w(W2)
    add_w(p['b2'].reshape(1, F))
    if use_ln:
        add_w(p['g2'].reshape(1, F))
        add_w(p['be2'].reshape(1, F))

    body = functools.partial(_mlp_body, group_sizes, has_res, use_ln, d_real1)
    return pl.pallas_call(
        body,
        grid=(N // block_r,),
        in_specs=specs,
        out_specs=pl.BlockSpec((block_r, F), lambda i: (i, 0)),
        out_shape=jax.ShapeDtypeStruct((N, F), jnp.float32),
    )(*ins)


_NW = 32
_CH = 128


_GL = 3


def _gather2(src_tab, dst_tab, src_idx_p, dst_idx_p):
    Ep = src_idx_p.shape[0]
    cpw = Ep // _NW
    nch = cpw // _CH
    mesh = plsc.VectorSubcoreMesh(core_axis_name="c", subcore_axis_name="s", num_cores=2, num_subcores=16)
    out_t = (jax.ShapeDtypeStruct((Ep, 128), jnp.float32),
             jax.ShapeDtypeStruct((Ep, 128), jnp.float32))

    @functools.partial(
        pl.kernel, mesh=mesh, out_type=out_t,
        scratch_types=[
            pltpu.VMEM((cpw,), jnp.int32),
            pltpu.VMEM((cpw,), jnp.int32),
            [pltpu.VMEM((_CH, 128), jnp.float32) for _ in range(_GL)],
            [pltpu.VMEM((_CH, 128), jnp.float32) for _ in range(_GL)],
            [pltpu.SemaphoreType.DMA for _ in range(2 * _GL)],
        ])
    def gk(src_hbm, dst_hbm, sidx_hbm, didx_hbm, out_s, out_d,
           iva, ivb, rs, rd, sems):
        c = lax.axis_index("c")
        s = lax.axis_index("s")
        base = (s * 2 + c) * cpw
        pltpu.sync_copy(sidx_hbm.at[pl.ds(base, cpw)], iva)
        pltpu.sync_copy(didx_hbm.at[pl.ds(base, cpw)], ivb)

        def issue(g, b):
            iv_s = iva.at[pl.ds(g * _CH, _CH)]
            iv_d = ivb.at[pl.ds(g * _CH, _CH)]
            pltpu.async_copy(src_hbm.at[iv_s], rs[b], sems[2 * b])
            pltpu.async_copy(dst_hbm.at[iv_d], rd[b], sems[2 * b + 1])

        for b in range(_GL):
            if b < nch:
                issue(b, b)

        nsup = -(-nch // _GL)

        def sup(i, carry):
            for b in range(_GL):
                g = i * _GL + b

                @pl.when(g < nch)
                def _(g=g, b=b):
                    pltpu.make_async_copy(
                        src_hbm.at[iva.at[pl.ds(g * _CH, _CH)]],
                        rs[b], sems[2 * b]).wait()
                    off = base + g * _CH
                    pltpu.sync_copy(rs[b], out_s.at[pl.ds(off, _CH)])
                    pltpu.make_async_copy(
                        dst_hbm.at[ivb.at[pl.ds(g * _CH, _CH)]],
                        rd[b], sems[2 * b + 1]).wait()
                    pltpu.sync_copy(rd[b], out_d.at[pl.ds(off, _CH)])

                @pl.when(g + _GL < nch)
                def _(g=g, b=b):
                    issue(g + _GL, b)
            return carry

        lax.fori_loop(0, nsup, sup, 0)

    return gk(src_tab, dst_tab, src_idx_p, dst_idx_p)


def _scatter_sum(rows, idx_p, n_ranges, range_size):
    Ep = idx_p.shape[0]
    cpw = Ep // _NW
    nch = cpw // _CH
    n_pad = n_ranges * range_size
    buf_rows = -(-(range_size + 1) // 128) * 128
    zrows = 8
    wb = range_size // 8
    nwb = -(-wb // 16)
    mesh = plsc.VectorSubcoreMesh(core_axis_name="c", subcore_axis_name="s", num_cores=2, num_subcores=16)

    if n_ranges == 1:
        SL, SCH = 2, 128
    else:
        SL, SCH = 3, 64
    LA = SL - 1

    nch_s = cpw // SCH

    @functools.partial(
        pl.kernel, mesh=mesh,
        out_type=jax.ShapeDtypeStruct((2, n_pad, 128), jnp.float32),
        scratch_types=[
            [pltpu.VMEM((SCH,), jnp.int32) for _ in range(SL)],
            [pltpu.VMEM((SCH,), jnp.int32) for _ in range(SL)],
            [pltpu.VMEM((SCH, 128), jnp.float32) for _ in range(SL)],
            pltpu.VMEM((zrows, 128), jnp.float32),
            pltpu.VMEM_SHARED((buf_rows, 128), jnp.float32),
            [pltpu.SemaphoreType.DMA for _ in range(2 * SL)],
        ])
    def sk(rows_hbm, idx_hbm, out_hbm, ivr, riv, rv, zb, acc, sems):
        c = lax.axis_index("c")
        s = lax.axis_index("s")
        base = (s * 2 + c) * cpw

        z16 = jnp.zeros((16,), jnp.float32)
        for r in range(zrows):
            for l in range(8):
                zb[r, pl.ds(l * 16, 16)] = z16

        zstripe = buf_rows // 16

        def load(g, b):
            pltpu.async_copy(rows_hbm.at[pl.ds(base + g * SCH, SCH)],
                             rv[b], sems[2 * b])
            pltpu.async_copy(idx_hbm.at[pl.ds(base + g * SCH, SCH)],
                             ivr[b], sems[2 * b])

        def wait_load(g, b):
            pltpu.make_async_copy(rows_hbm.at[pl.ds(base + g * SCH, SCH)],
                                  rv[b], sems[2 * b]).wait()
            pltpu.make_async_copy(idx_hbm.at[pl.ds(base + g * SCH, SCH)],
                                  ivr[b], sems[2 * b]).wait()

        def wait_scat(b):
            pltpu.make_async_copy(rv[b], acc.at[riv[b]],
                                  sems[2 * b + 1]).wait()

        for rg in range(n_ranges):
            lo = rg * range_size

            def zbody(i, carry):
                acc_off = s * zstripe + i * zrows
                pltpu.sync_copy(zb, acc.at[pl.ds(acc_off, zrows)])
                return carry
            lax.fori_loop(0, zstripe // zrows, zbody, 0)
            plsc.subcore_barrier()

            for b in range(LA):
                if b < nch_s:
                    load(b, b)

            nsup = -(-nch_s // SL)

            def sup(i, carry):
                for b in range(SL):
                    g = i * SL + b

                    @pl.when(g < nch_s)
                    def _(g=g, b=b):
                        wait_load(g, b)
                        for t in range(SCH // 16):
                            v = ivr[b][pl.ds(t * 16, 16)]
                            ok = (v >= lo) & (v < lo + range_size)
                            riv[b][pl.ds(t * 16, 16)] = jnp.where(
                                ok, v - lo, range_size)
                        pltpu.async_copy(rv[b], acc.at[riv[b]],
                                         sems[2 * b + 1], add=True)
                        b2 = (b + LA) % SL
                        g2 = g + LA

                        @pl.when(g2 < nch_s)
                        def _():
                            @pl.when(g2 - SL >= 0)
                            def _():
                                wait_scat(b2)
                            load(g2, b2)
                return carry

            lax.fori_loop(0, nsup, sup, 0)

            for tail in range(max(nch_s - SL, 0), nch_s):
                wait_scat(tail % SL)
            plsc.subcore_barrier()

            def wbody(i, carry):
                t = i * 16 + s

                @pl.when(t < wb)
                def _():
                    pltpu.sync_copy(acc.at[pl.ds(t * 8, 8)],
                                    out_hbm.at[c, pl.ds(lo + t * 8, 8)])
                return carry
            lax.fori_loop(0, nwb, wbody, 0)
            plsc.subcore_barrier()

    return sk(rows, idx_p)


def _pad_e(E):
    return ((E + 4095) // 4096) * 4096


def _pad_idx(idx, Ep, fill):
    return jnp.pad(idx, (0, Ep - idx.shape[0]), constant_values=fill)


def _edge_embed(efeat, p, Ep):
    E = efeat.shape[0]
    x = jnp.zeros((Ep, 8), jnp.float32).at[:E, :4].set(efeat)
    W1p = jnp.zeros((8, 8), jnp.float32).at[:4, :4].set(p['W1'])
    W2p = jnp.zeros((8, 128), jnp.float32).at[:4, :].set(p['W2'])
    pp = {
        'W1': W1p,
        'b1': jnp.zeros((8,), jnp.float32).at[:4].set(p['b1']),
        'g1': jnp.zeros((8,), jnp.float32).at[:4].set(p['g1']),
        'be1': jnp.zeros((8,), jnp.float32).at[:4].set(p['be1']),
        'W2': W2p,
        'b2': p['b2'],
        'g2': p['g2'],
        'be2': p['be2'],
    }
    return _mlp([[x]], pp, 512, d_real1=4)


def _gnn_step(pg, src_h, dst_h, e_parts, src_idx, dst_idx, n_dst, n_ranges,
              block_dst, same_type, need_src_update, need_e_out):
    E = src_idx.shape[0]
    Ep = _pad_e(E)
    sidx = _pad_idx(src_idx, Ep, 0)
    didx = _pad_idx(dst_idx, Ep, 0)
    didx_s = _pad_idx(dst_idx, Ep, -1)

    gs, gd = _gather2(src_h, dst_h, sidx, didx)
    e_new = _mlp([e_parts, [gs], [gd]], pg['edge'], 512)

    range_size = -(-n_dst // (8 * n_ranges)) * 8
    p = _scatter_sum(e_new, didx_s, n_ranges, range_size)
    p0, p1 = p[0], p[1]

    dst2 = _mlp([[dst_h], [p0, p1]], pg['dst'], block_dst, residual=dst_h)

    if need_src_update:
        base = dst2 if same_type else src_h
        src2 = _mlp([[base]], pg['src'], block_dst if same_type else 400,
                    residual=base)
    else:
        src2 = None
    e_out = (e_parts, e_new) if need_e_out else None
    return src2, dst2, e_out


def kernel(params, weather_grid_feat, grid_static, mesh_static,
           g2m_efeat, m2m_efeat, m2g_efeat,
           g2m_src, g2m_dst, m2m_src, m2m_dst, m2g_src, m2g_dst):
    P = params
    w = weather_grid_feat[0]

    grid_h = _mlp([[w], [grid_static]], P['grid_embed'], 400)
    me = P['mesh_embed']
    mesh_h = _mlp([[mesh_static]], me, 400, W1=me['W1'][64:])

    Eg = _pad_e(E_G2M)
    Em = _pad_e(E_M2M)
    Eo = _pad_e(E_M2G)
    g2m_e = _edge_embed(g2m_efeat, P['g2m_edge_embed'], Eg)
    m2m_e = _edge_embed(m2m_efeat, P['m2m_edge_embed'], Em)
    m2g_e = _edge_embed(m2g_efeat, P['m2g_edge_embed'], Eo)

    grid_h, mesh_h, _ = _gnn_step(
        P['g2m_gnn'], grid_h, mesh_h, [g2m_e], g2m_src, g2m_dst,
        N_MESH, 1, 400, same_type=False, need_src_update=True,
        need_e_out=False)

    mesh_h2, _, e_carry = _gnn_step(
        P['m2m_gnns'][0], mesh_h, mesh_h, [m2m_e], m2m_src, m2m_dst,
        N_MESH, 1, 400, same_type=True, need_src_update=True,
        need_e_out=True)
    mesh_h = mesh_h2
    e_parts = list(e_carry[0]) + [e_carry[1]]
    mesh_h2, _, _ = _gnn_step(
        P['m2m_gnns'][1], mesh_h, mesh_h, e_parts, m2m_src, m2m_dst,
        N_MESH, 1, 400, same_type=True, need_src_update=True,
        need_e_out=False)
    mesh_h = mesh_h2

    _, grid_h, _ = _gnn_step(
        P['m2g_gnn'], mesh_h, grid_h, [m2g_e], m2g_src, m2g_dst,
        N_GRID, 4, 400, same_type=False, need_src_update=False,
        need_e_out=False)

    out = _mlp([[grid_h]], P['m2g_out'], 400, use_ln=False)
    return out[:, None, :]


# speedup vs baseline: 1.0142x; 1.0142x over previous
import functools

import jax
import jax.numpy as jnp
from jax import lax
from jax.experimental import pallas as pl
from jax.experimental.pallas import tpu as pltpu
from jax.experimental.pallas import tpu_sc as plsc

_EPS = 1e-5

N_GRID = 50000
N_MESH = 10000
E_G2M = 100000
E_M2M = 100000
E_M2G = 150000


def _mlp_body(group_sizes, has_res, use_ln, d_real1, *refs):
    idx = 0
    xs = []
    for g in group_sizes:
        acc = refs[idx][...].astype(jnp.float32)
        for t in range(1, g):
            acc = acc + refs[idx + t][...].astype(jnp.float32)
        idx += g
        xs.append(acc)
    x = xs[0] if len(xs) == 1 else jnp.concatenate(xs, axis=-1)
    res = None
    if has_res:
        res = refs[idx][...]
        idx += 1
    W1 = refs[idx][...]
    b1 = refs[idx + 1][...]
    idx += 2
    if use_ln:
        g1 = refs[idx][...]
        be1 = refs[idx + 1][...]
        idx += 2
    W2 = refs[idx][...]
    b2 = refs[idx + 1][...]
    idx += 2
    if use_ln:
        g2 = refs[idx][...]
        be2 = refs[idx + 1][...]
        idx += 2
    out_ref = refs[idx]

    h = jnp.dot(x, W1, preferred_element_type=jnp.float32) + b1
    if use_ln:
        H = h.shape[-1]
        if d_real1 == H:
            m = jnp.mean(h, axis=-1, keepdims=True)
            hc = h - m
        else:
            m = jnp.sum(h, axis=-1, keepdims=True) / d_real1
            mask = lax.broadcasted_iota(jnp.int32, h.shape, 1) < d_real1
            hc = jnp.where(mask, h - m, 0.0)
        v = jnp.sum(hc * hc, axis=-1, keepdims=True) / d_real1
        h = hc * lax.rsqrt(v + _EPS) * g1 + be1
    h = h * (1.0 / (1.0 + jnp.exp(-h)))
    y = jnp.dot(h, W2, preferred_element_type=jnp.float32) + b2
    if use_ln:
        m2 = jnp.mean(y, axis=-1, keepdims=True)
        yc = y - m2
        v2 = jnp.mean(yc * yc, axis=-1, keepdims=True)
        y = yc * lax.rsqrt(v2 + _EPS) * g2 + be2
    if has_res:
        y = y + res
    out_ref[...] = y


def _mlp(parts, p, block_r, *, residual=None, use_ln=True, d_real1=None,
         W1=None):
    N = parts[0][0].shape[0]
    assert N % block_r == 0, (N, block_r)
    W1 = p['W1'] if W1 is None else W1
    W2 = p['W2']
    H = W1.shape[1]
    F = W2.shape[1]
    if d_real1 is None:
        d_real1 = H
    group_sizes = tuple(len(g) for g in parts)
    flat = [a for g in parts for a in g]

    ins = []
    specs = []
    for a in flat:
        ins.append(a)
        specs.append(pl.BlockSpec((block_r, a.shape[1]), lambda i: (i, 0)))
    has_res = residual is not None
    if has_res:
        ins.append(residual)
        specs.append(pl.BlockSpec((block_r, F), lambda i: (i, 0)))

    def add_w(w):
        ins.append(w)
        specs.append(pl.BlockSpec(w.shape, lambda i: (0,) * w.ndim))

    add_w(W1)
    add_w(p['b1'].reshape(1, H))
    if use_ln:
        add_w(p['g1'].reshape(1, H))
        add_w(p['be1'].reshape(1, H))
    add_w(W2)
    add_w(p['b2'].reshape(1, F))
    if use_ln:
        add_w(p['g2'].reshape(1, F))
        add_w(p['be2'].reshape(1, F))

    body = functools.partial(_mlp_body, group_sizes, has_res, use_ln, d_real1)
    return pl.pallas_call(
        body,
        grid=(N // block_r,),
        in_specs=specs,
        out_specs=pl.BlockSpec((block_r, F), lambda i: (i, 0)),
        out_shape=jax.ShapeDtypeStruct((N, F), jnp.float32),
    )(*ins)


_NW = 32
_CH = 128


_GL = 3


def _gather2(src_tab, dst_tab, src_idx_p, dst_idx_p):
    Ep = src_idx_p.shape[0]
    cpw = Ep // _NW
    nch = cpw // _CH
    mesh = plsc.VectorSubcoreMesh(core_axis_name="c", subcore_axis_name="s", num_cores=2, num_subcores=16)
    out_t = (jax.ShapeDtypeStruct((Ep, 128), jnp.float32),
             jax.ShapeDtypeStruct((Ep, 128), jnp.float32))

    @functools.partial(
        pl.kernel, mesh=mesh, out_type=out_t,
        scratch_types=[
            pltpu.VMEM((cpw,), jnp.int32),
            pltpu.VMEM((cpw,), jnp.int32),
            [pltpu.VMEM((_CH, 128), jnp.float32) for _ in range(_GL)],
            [pltpu.VMEM((_CH, 128), jnp.float32) for _ in range(_GL)],
            [pltpu.SemaphoreType.DMA for _ in range(4 * _GL)],
        ])
    def gk(src_hbm, dst_hbm, sidx_hbm, didx_hbm, out_s, out_d,
           iva, ivb, rs, rd, sems):
        c = lax.axis_index("c")
        s = lax.axis_index("s")
        base = (s * 2 + c) * cpw
        pltpu.sync_copy(sidx_hbm.at[pl.ds(base, cpw)], iva)
        pltpu.sync_copy(didx_hbm.at[pl.ds(base, cpw)], ivb)

        def issue(g, b):
            iv_s = iva.at[pl.ds(g * _CH, _CH)]
            iv_d = ivb.at[pl.ds(g * _CH, _CH)]
            pltpu.async_copy(src_hbm.at[iv_s], rs[b], sems[4 * b])
            pltpu.async_copy(dst_hbm.at[iv_d], rd[b], sems[4 * b + 1])

        def wait_wb(g, b):
            off = base + g * _CH
            pltpu.make_async_copy(rs[b], out_s.at[pl.ds(off, _CH)],
                                  sems[4 * b + 2]).wait()
            pltpu.make_async_copy(rd[b], out_d.at[pl.ds(off, _CH)],
                                  sems[4 * b + 3]).wait()

        for b in range(_GL - 1):
            if b < nch:
                issue(b, b)

        nsup = -(-nch // _GL)

        def sup(i, carry):
            for b in range(_GL):
                g = i * _GL + b

                @pl.when(g < nch)
                def _(g=g, b=b):
                    off = base + g * _CH
                    pltpu.make_async_copy(
                        src_hbm.at[iva.at[pl.ds(g * _CH, _CH)]],
                        rs[b], sems[4 * b]).wait()
                    pltpu.async_copy(rs[b], out_s.at[pl.ds(off, _CH)],
                                     sems[4 * b + 2])
                    pltpu.make_async_copy(
                        dst_hbm.at[ivb.at[pl.ds(g * _CH, _CH)]],
                        rd[b], sems[4 * b + 1]).wait()
                    pltpu.async_copy(rd[b], out_d.at[pl.ds(off, _CH)],
                                     sems[4 * b + 3])
                    b2 = (b + _GL - 1) % _GL
                    g2 = g + _GL - 1

                    @pl.when(g2 < nch)
                    def _():
                        @pl.when(g2 - _GL >= 0)
                        def _():
                            wait_wb(g2 - _GL, b2)
                        issue(g2, b2)
            return carry

        lax.fori_loop(0, nsup, sup, 0)

        for tail in range(max(nch - _GL, 0), nch):
            wait_wb(tail, tail % _GL)

    return gk(src_tab, dst_tab, src_idx_p, dst_idx_p)


def _scatter_sum(rows, idx_p, n_ranges, range_size):
    Ep = idx_p.shape[0]
    cpw = Ep // _NW
    nch = cpw // _CH
    n_pad = n_ranges * range_size
    buf_rows = -(-(range_size + 1) // 128) * 128
    zrows = 8
    wb = range_size // 8
    nwb = -(-wb // 16)
    mesh = plsc.VectorSubcoreMesh(core_axis_name="c", subcore_axis_name="s", num_cores=2, num_subcores=16)

    if n_ranges == 1:
        SL, SCH = 2, 128
    else:
        SL, SCH = 3, 64
    LA = SL - 1

    nch_s = cpw // SCH

    @functools.partial(
        pl.kernel, mesh=mesh,
        out_type=jax.ShapeDtypeStruct((2, n_pad, 128), jnp.float32),
        scratch_types=[
            [pltpu.VMEM((SCH,), jnp.int32) for _ in range(SL)],
            [pltpu.VMEM((SCH,), jnp.int32) for _ in range(SL)],
            [pltpu.VMEM((SCH, 128), jnp.float32) for _ in range(SL)],
            pltpu.VMEM((zrows, 128), jnp.float32),
            pltpu.VMEM_SHARED((buf_rows, 128), jnp.float32),
            [pltpu.SemaphoreType.DMA for _ in range(2 * SL)],
        ])
    def sk(rows_hbm, idx_hbm, out_hbm, ivr, riv, rv, zb, acc, sems):
        c = lax.axis_index("c")
        s = lax.axis_index("s")
        base = (s * 2 + c) * cpw

        z16 = jnp.zeros((16,), jnp.float32)
        for r in range(zrows):
            for l in range(8):
                zb[r, pl.ds(l * 16, 16)] = z16

        zstripe = buf_rows // 16

        def load(g, b):
            pltpu.async_copy(rows_hbm.at[pl.ds(base + g * SCH, SCH)],
                             rv[b], sems[2 * b])
            pltpu.async_copy(idx_hbm.at[pl.ds(base + g * SCH, SCH)],
                             ivr[b], sems[2 * b])

        def wait_load(g, b):
            pltpu.make_async_copy(rows_hbm.at[pl.ds(base + g * SCH, SCH)],
                                  rv[b], sems[2 * b]).wait()
            pltpu.make_async_copy(idx_hbm.at[pl.ds(base + g * SCH, SCH)],
                                  ivr[b], sems[2 * b]).wait()

        def wait_scat(b):
            pltpu.make_async_copy(rv[b], acc.at[riv[b]],
                                  sems[2 * b + 1]).wait()

        for rg in range(n_ranges):
            lo = rg * range_size

            def zbody(i, carry):
                acc_off = s * zstripe + i * zrows
                pltpu.sync_copy(zb, acc.at[pl.ds(acc_off, zrows)])
                return carry
            lax.fori_loop(0, zstripe // zrows, zbody, 0)
            plsc.subcore_barrier()

            for b in range(LA):
                if b < nch_s:
                    load(b, b)

            nsup = -(-nch_s // SL)

            def sup(i, carry):
                for b in range(SL):
                    g = i * SL + b

                    @pl.when(g < nch_s)
                    def _(g=g, b=b):
                        wait_load(g, b)
                        for t in range(SCH // 16):
                            v = ivr[b][pl.ds(t * 16, 16)]
                            ok = (v >= lo) & (v < lo + range_size)
                            riv[b][pl.ds(t * 16, 16)] = jnp.where(
                                ok, v - lo, range_size)
                        pltpu.async_copy(rv[b], acc.at[riv[b]],
                                         sems[2 * b + 1], add=True)
                        b2 = (b + LA) % SL
                        g2 = g + LA

                        @pl.when(g2 < nch_s)
                        def _():
                            @pl.when(g2 - SL >= 0)
                            def _():
                                wait_scat(b2)
                            load(g2, b2)
                return carry

            lax.fori_loop(0, nsup, sup, 0)

            for tail in range(max(nch_s - SL, 0), nch_s):
                wait_scat(tail % SL)
            plsc.subcore_barrier()

            def wbody(i, carry):
                t = i * 16 + s

                @pl.when(t < wb)
                def _():
                    pltpu.sync_copy(acc.at[pl.ds(t * 8, 8)],
                                    out_hbm.at[c, pl.ds(lo + t * 8, 8)])
                return carry
            lax.fori_loop(0, nwb, wbody, 0)
            plsc.subcore_barrier()

    return sk(rows, idx_p)


def _pad_e(E):
    return ((E + 4095) // 4096) * 4096


def _pad_idx(idx, Ep, fill):
    return jnp.pad(idx, (0, Ep - idx.shape[0]), constant_values=fill)


def _edge_embed(efeat, p, Ep):
    E = efeat.shape[0]
    x = jnp.zeros((Ep, 8), jnp.float32).at[:E, :4].set(efeat)
    W1p = jnp.zeros((8, 8), jnp.float32).at[:4, :4].set(p['W1'])
    W2p = jnp.zeros((8, 128), jnp.float32).at[:4, :].set(p['W2'])
    pp = {
        'W1': W1p,
        'b1': jnp.zeros((8,), jnp.float32).at[:4].set(p['b1']),
        'g1': jnp.zeros((8,), jnp.float32).at[:4].set(p['g1']),
        'be1': jnp.zeros((8,), jnp.float32).at[:4].set(p['be1']),
        'W2': W2p,
        'b2': p['b2'],
        'g2': p['g2'],
        'be2': p['be2'],
    }
    return _mlp([[x]], pp, 512, d_real1=4)


def _gnn_step(pg, src_h, dst_h, e_parts, src_idx, dst_idx, n_dst, n_ranges,
              block_dst, same_type, need_src_update, need_e_out):
    E = src_idx.shape[0]
    Ep = _pad_e(E)
    sidx = _pad_idx(src_idx, Ep, 0)
    didx = _pad_idx(dst_idx, Ep, 0)
    didx_s = _pad_idx(dst_idx, Ep, -1)

    gs, gd = _gather2(src_h, dst_h, sidx, didx)
    e_new = _mlp([e_parts, [gs], [gd]], pg['edge'], 512)

    range_size = -(-n_dst // (8 * n_ranges)) * 8
    p = _scatter_sum(e_new, didx_s, n_ranges, range_size)
    p0, p1 = p[0], p[1]

    dst2 = _mlp([[dst_h], [p0, p1]], pg['dst'], block_dst, residual=dst_h)

    if need_src_update:
        base = dst2 if same_type else src_h
        src2 = _mlp([[base]], pg['src'], block_dst if same_type else 400,
                    residual=base)
    else:
        src2 = None
    e_out = (e_parts, e_new) if need_e_out else None
    return src2, dst2, e_out


def kernel(params, weather_grid_feat, grid_static, mesh_static,
           g2m_efeat, m2m_efeat, m2g_efeat,
           g2m_src, g2m_dst, m2m_src, m2m_dst, m2g_src, m2g_dst):
    P = params
    w = weather_grid_feat[0]

    grid_h = _mlp([[w], [grid_static]], P['grid_embed'], 400)
    me = P['mesh_embed']
    mesh_h = _mlp([[mesh_static]], me, 400, W1=me['W1'][64:])

    Eg = _pad_e(E_G2M)
    Em = _pad_e(E_M2M)
    Eo = _pad_e(E_M2G)
    g2m_e = _edge_embed(g2m_efeat, P['g2m_edge_embed'], Eg)
    m2m_e = _edge_embed(m2m_efeat, P['m2m_edge_embed'], Em)
    m2g_e = _edge_embed(m2g_efeat, P['m2g_edge_embed'], Eo)

    grid_h, mesh_h, _ = _gnn_step(
        P['g2m_gnn'], grid_h, mesh_h, [g2m_e], g2m_src, g2m_dst,
        N_MESH, 1, 400, same_type=False, need_src_update=True,
        need_e_out=False)

    mesh_h2, _, e_carry = _gnn_step(
        P['m2m_gnns'][0], mesh_h, mesh_h, [m2m_e], m2m_src, m2m_dst,
        N_MESH, 1, 400, same_type=True, need_src_update=True,
        need_e_out=True)
    mesh_h = mesh_h2
    e_parts = list(e_carry[0]) + [e_carry[1]]
    mesh_h2, _, _ = _gnn_step(
        P['m2m_gnns'][1], mesh_h, mesh_h, e_parts, m2m_src, m2m_dst,
        N_MESH, 1, 400, same_type=True, need_src_update=True,
        need_e_out=False)
    mesh_h = mesh_h2

    _, grid_h, _ = _gnn_step(
        P['m2g_gnn'], mesh_h, grid_h, [m2g_e], m2g_src, m2g_dst,
        N_GRID, 4, 400, same_type=False, need_src_update=False,
        need_e_out=False)

    out = _mlp([[grid_h]], P['m2g_out'], 400, use_ln=False)
    return out[:, None, :]


# speedup vs baseline: 1.0171x; 1.0028x over previous
import functools

import jax
import jax.numpy as jnp
from jax import lax
from jax.experimental import pallas as pl
from jax.experimental.pallas import tpu as pltpu
from jax.experimental.pallas import tpu_sc as plsc

_EPS = 1e-5

N_GRID = 50000
N_MESH = 10000
E_G2M = 100000
E_M2M = 100000
E_M2G = 150000


def _mlp_body(group_sizes, has_res, use_ln, d_real1, *refs):
    idx = 0
    xs = []
    for g in group_sizes:
        acc = refs[idx][...].astype(jnp.float32)
        for t in range(1, g):
            acc = acc + refs[idx + t][...].astype(jnp.float32)
        idx += g
        xs.append(acc)
    x = xs[0] if len(xs) == 1 else jnp.concatenate(xs, axis=-1)
    res = None
    if has_res:
        res = refs[idx][...]
        idx += 1
    W1 = refs[idx][...]
    b1 = refs[idx + 1][...]
    idx += 2
    if use_ln:
        g1 = refs[idx][...]
        be1 = refs[idx + 1][...]
        idx += 2
    W2 = refs[idx][...]
    b2 = refs[idx + 1][...]
    idx += 2
    if use_ln:
        g2 = refs[idx][...]
        be2 = refs[idx + 1][...]
        idx += 2
    out_ref = refs[idx]

    h = jnp.dot(x, W1, preferred_element_type=jnp.float32) + b1
    if use_ln:
        H = h.shape[-1]
        if d_real1 == H:
            m = jnp.mean(h, axis=-1, keepdims=True)
            hc = h - m
        else:
            m = jnp.sum(h, axis=-1, keepdims=True) / d_real1
            mask = lax.broadcasted_iota(jnp.int32, h.shape, 1) < d_real1
            hc = jnp.where(mask, h - m, 0.0)
        v = jnp.sum(hc * hc, axis=-1, keepdims=True) / d_real1
        h = hc * lax.rsqrt(v + _EPS) * g1 + be1
    h = h * (1.0 / (1.0 + jnp.exp(-h)))
    y = jnp.dot(h, W2, preferred_element_type=jnp.float32) + b2
    if use_ln:
        m2 = jnp.mean(y, axis=-1, keepdims=True)
        yc = y - m2
        v2 = jnp.mean(yc * yc, axis=-1, keepdims=True)
        y = yc * lax.rsqrt(v2 + _EPS) * g2 + be2
    if has_res:
        y = y + res
    out_ref[...] = y


def _mlp(parts, p, block_r, *, residual=None, use_ln=True, d_real1=None,
         W1=None):
    N = parts[0][0].shape[0]
    assert N % block_r == 0, (N, block_r)
    W1 = p['W1'] if W1 is None else W1
    W2 = p['W2']
    H = W1.shape[1]
    F = W2.shape[1]
    if d_real1 is None:
        d_real1 = H
    group_sizes = tuple(len(g) for g in parts)
    flat = [a for g in parts for a in g]

    ins = []
    specs = []
    for a in flat:
        ins.append(a)
        specs.append(pl.BlockSpec((block_r, a.shape[1]), lambda i: (i, 0)))
    has_res = residual is not None
    if has_res:
        ins.append(residual)
        specs.append(pl.BlockSpec((block_r, F), lambda i: (i, 0)))

    def add_w(w):
        ins.append(w)
        specs.append(pl.BlockSpec(w.shape, lambda i: (0,) * w.ndim))

    add_w(W1)
    add_w(p['b1'].reshape(1, H))
    if use_ln:
        add_w(p['g1'].reshape(1, H))
        add_w(p['be1'].reshape(1, H))
    add_w(W2)
    add_w(p['b2'].reshape(1, F))
    if use_ln:
        add_w(p['g2'].reshape(1, F))
        add_w(p['be2'].reshape(1, F))

    body = functools.partial(_mlp_body, group_sizes, has_res, use_ln, d_real1)
    return pl.pallas_call(
        body,
        grid=(N // block_r,),
        in_specs=specs,
        out_specs=pl.BlockSpec((block_r, F), lambda i: (i, 0)),
        out_shape=jax.ShapeDtypeStruct((N, F), jnp.float32),
    )(*ins)


_NW = 32
_CH = 128


_GL = 3


def _gather2(src_tab, dst_tab, src_idx_p, dst_idx_p):
    Ep = src_idx_p.shape[0]
    cpw = Ep // _NW
    nch = cpw // _CH
    mesh = plsc.VectorSubcoreMesh(core_axis_name="c", subcore_axis_name="s", num_cores=2, num_subcores=16)
    out_t = (jax.ShapeDtypeStruct((Ep, 128), jnp.float32),
             jax.ShapeDtypeStruct((Ep, 128), jnp.float32))

    @functools.partial(
        pl.kernel, mesh=mesh, out_type=out_t,
        scratch_types=[
            pltpu.VMEM((cpw,), jnp.int32),
            pltpu.VMEM((cpw,), jnp.int32),
            [pltpu.VMEM((_CH, 128), jnp.float32) for _ in range(_GL)],
            [pltpu.VMEM((_CH, 128), jnp.float32) for _ in range(_GL)],
            [pltpu.SemaphoreType.DMA for _ in range(4 * _GL)],
        ])
    def gk(src_hbm, dst_hbm, sidx_hbm, didx_hbm, out_s, out_d,
           iva, ivb, rs, rd, sems):
        c = lax.axis_index("c")
        s = lax.axis_index("s")
        base = (s * 2 + c) * cpw
        pltpu.sync_copy(sidx_hbm.at[pl.ds(base, cpw)], iva)
        pltpu.sync_copy(didx_hbm.at[pl.ds(base, cpw)], ivb)

        def issue(g, b):
            iv_s = iva.at[pl.ds(g * _CH, _CH)]
            iv_d = ivb.at[pl.ds(g * _CH, _CH)]
            pltpu.async_copy(src_hbm.at[iv_s], rs[b], sems[4 * b])
            pltpu.async_copy(dst_hbm.at[iv_d], rd[b], sems[4 * b + 1])

        def wait_wb(g, b):
            off = base + g * _CH
            pltpu.make_async_copy(rs[b], out_s.at[pl.ds(off, _CH)],
                                  sems[4 * b + 2]).wait()
            pltpu.make_async_copy(rd[b], out_d.at[pl.ds(off, _CH)],
                                  sems[4 * b + 3]).wait()

        for b in range(_GL - 1):
            if b < nch:
                issue(b, b)

        nsup = -(-nch // _GL)

        def sup(i, carry):
            for b in range(_GL):
                g = i * _GL + b

                @pl.when(g < nch)
                def _(g=g, b=b):
                    off = base + g * _CH
                    pltpu.make_async_copy(
                        src_hbm.at[iva.at[pl.ds(g * _CH, _CH)]],
                        rs[b], sems[4 * b]).wait()
                    pltpu.async_copy(rs[b], out_s.at[pl.ds(off, _CH)],
                                     sems[4 * b + 2])
                    pltpu.make_async_copy(
                        dst_hbm.at[ivb.at[pl.ds(g * _CH, _CH)]],
                        rd[b], sems[4 * b + 1]).wait()
                    pltpu.async_copy(rd[b], out_d.at[pl.ds(off, _CH)],
                                     sems[4 * b + 3])
                    b2 = (b + _GL - 1) % _GL
                    g2 = g + _GL - 1

                    @pl.when(g2 < nch)
                    def _():
                        @pl.when(g2 - _GL >= 0)
                        def _():
                            wait_wb(g2 - _GL, b2)
                        issue(g2, b2)
            return carry

        lax.fori_loop(0, nsup, sup, 0)

        for tail in range(max(nch - _GL, 0), nch):
            wait_wb(tail, tail % _GL)

    return gk(src_tab, dst_tab, src_idx_p, dst_idx_p)


def _scatter_sum(rows, idx_p, n_ranges, range_size):
    Ep = idx_p.shape[0]
    cpw = Ep // _NW
    nch = cpw // _CH
    n_pad = n_ranges * range_size
    buf_rows = -(-(range_size + 1) // 128) * 128
    zrows = 8
    wb = range_size // 8
    nwb = -(-wb // 16)
    mesh = plsc.VectorSubcoreMesh(core_axis_name="c", subcore_axis_name="s", num_cores=2, num_subcores=16)

    SL, SCH = 3, 64
    LA = SL - 1

    nch_s = cpw // SCH

    @functools.partial(
        pl.kernel, mesh=mesh,
        out_type=jax.ShapeDtypeStruct((2, n_pad, 128), jnp.float32),
        scratch_types=[
            [pltpu.VMEM((SCH,), jnp.int32) for _ in range(SL)],
            [pltpu.VMEM((SCH,), jnp.int32) for _ in range(SL)],
            [pltpu.VMEM((SCH, 128), jnp.float32) for _ in range(SL)],
            pltpu.VMEM((zrows, 128), jnp.float32),
            pltpu.VMEM_SHARED((buf_rows, 128), jnp.float32),
            [pltpu.SemaphoreType.DMA for _ in range(2 * SL)],
        ])
    def sk(rows_hbm, idx_hbm, out_hbm, ivr, riv, rv, zb, acc, sems):
        c = lax.axis_index("c")
        s = lax.axis_index("s")
        base = (s * 2 + c) * cpw

        z16 = jnp.zeros((16,), jnp.float32)
        for r in range(zrows):
            for l in range(8):
                zb[r, pl.ds(l * 16, 16)] = z16

        zstripe = buf_rows // 16

        def load(g, b):
            pltpu.async_copy(rows_hbm.at[pl.ds(base + g * SCH, SCH)],
                             rv[b], sems[2 * b])
            pltpu.async_copy(idx_hbm.at[pl.ds(base + g * SCH, SCH)],
                             ivr[b], sems[2 * b])

        def wait_load(g, b):
            pltpu.make_async_copy(rows_hbm.at[pl.ds(base + g * SCH, SCH)],
                                  rv[b], sems[2 * b]).wait()
            pltpu.make_async_copy(idx_hbm.at[pl.ds(base + g * SCH, SCH)],
                                  ivr[b], sems[2 * b]).wait()

        def wait_scat(b):
            pltpu.make_async_copy(rv[b], acc.at[riv[b]],
                                  sems[2 * b + 1]).wait()

        for rg in range(n_ranges):
            lo = rg * range_size

            def zbody(i, carry):
                acc_off = s * zstripe + i * zrows
                pltpu.sync_copy(zb, acc.at[pl.ds(acc_off, zrows)])
                return carry
            lax.fori_loop(0, zstripe // zrows, zbody, 0)
            plsc.subcore_barrier()

            for b in range(LA):
                if b < nch_s:
                    load(b, b)

            nsup = -(-nch_s // SL)

            def sup(i, carry):
                for b in range(SL):
                    g = i * SL + b

                    @pl.when(g < nch_s)
                    def _(g=g, b=b):
                        wait_load(g, b)
                        for t in range(SCH // 16):
                            v = ivr[b][pl.ds(t * 16, 16)]
                            ok = (v >= lo) & (v < lo + range_size)
                            riv[b][pl.ds(t * 16, 16)] = jnp.where(
                                ok, v - lo, range_size)
                        pltpu.async_copy(rv[b], acc.at[riv[b]],
                                         sems[2 * b + 1], add=True)
                        b2 = (b + LA) % SL
                        g2 = g + LA

                        @pl.when(g2 < nch_s)
                        def _():
                            @pl.when(g2 - SL >= 0)
                            def _():
                                wait_scat(b2)
                            load(g2, b2)
                return carry

            lax.fori_loop(0, nsup, sup, 0)

            for tail in range(max(nch_s - SL, 0), nch_s):
                wait_scat(tail % SL)
            plsc.subcore_barrier()

            def wbody(i, carry):
                t = i * 16 + s

                @pl.when(t < wb)
                def _():
                    pltpu.sync_copy(acc.at[pl.ds(t * 8, 8)],
                                    out_hbm.at[c, pl.ds(lo + t * 8, 8)])
                return carry
            lax.fori_loop(0, nwb, wbody, 0)
            plsc.subcore_barrier()

    return sk(rows, idx_p)


def _pad_e(E):
    return ((E + 4095) // 4096) * 4096


def _pad_idx(idx, Ep, fill):
    return jnp.pad(idx, (0, Ep - idx.shape[0]), constant_values=fill)


def _edge_embed(efeat, p, Ep):
    E = efeat.shape[0]
    x = jnp.zeros((Ep, 8), jnp.float32).at[:E, :4].set(efeat)
    W1p = jnp.zeros((8, 8), jnp.float32).at[:4, :4].set(p['W1'])
    W2p = jnp.zeros((8, 128), jnp.float32).at[:4, :].set(p['W2'])
    pp = {
        'W1': W1p,
        'b1': jnp.zeros((8,), jnp.float32).at[:4].set(p['b1']),
        'g1': jnp.zeros((8,), jnp.float32).at[:4].set(p['g1']),
        'be1': jnp.zeros((8,), jnp.float32).at[:4].set(p['be1']),
        'W2': W2p,
        'b2': p['b2'],
        'g2': p['g2'],
        'be2': p['be2'],
    }
    return _mlp([[x]], pp, 512, d_real1=4)


def _gnn_step(pg, src_h, dst_h, e_parts, src_idx, dst_idx, n_dst, n_ranges,
              block_dst, same_type, need_src_update, need_e_out):
    E = src_idx.shape[0]
    Ep = _pad_e(E)
    sidx = _pad_idx(src_idx, Ep, 0)
    didx = _pad_idx(dst_idx, Ep, 0)
    didx_s = _pad_idx(dst_idx, Ep, -1)

    gs, gd = _gather2(src_h, dst_h, sidx, didx)
    e_new = _mlp([e_parts, [gs], [gd]], pg['edge'], 512)

    range_size = -(-n_dst // (8 * n_ranges)) * 8
    p = _scatter_sum(e_new, didx_s, n_ranges, range_size)
    p0, p1 = p[0], p[1]

    dst2 = _mlp([[dst_h], [p0, p1]], pg['dst'], block_dst, residual=dst_h)

    if need_src_update:
        base = dst2 if same_type else src_h
        src2 = _mlp([[base]], pg['src'], block_dst if same_type else 400,
                    residual=base)
    else:
        src2 = None
    e_out = (e_parts, e_new) if need_e_out else None
    return src2, dst2, e_out


def kernel(params, weather_grid_feat, grid_static, mesh_static,
           g2m_efeat, m2m_efeat, m2g_efeat,
           g2m_src, g2m_dst, m2m_src, m2m_dst, m2g_src, m2g_dst):
    P = params
    w = weather_grid_feat[0]

    grid_h = _mlp([[w], [grid_static]], P['grid_embed'], 400)
    me = P['mesh_embed']
    mesh_h = _mlp([[mesh_static]], me, 400, W1=me['W1'][64:])

    Eg = _pad_e(E_G2M)
    Em = _pad_e(E_M2M)
    Eo = _pad_e(E_M2G)
    g2m_e = _edge_embed(g2m_efeat, P['g2m_edge_embed'], Eg)
    m2m_e = _edge_embed(m2m_efeat, P['m2m_edge_embed'], Em)
    m2g_e = _edge_embed(m2g_efeat, P['m2g_edge_embed'], Eo)

    grid_h, mesh_h, _ = _gnn_step(
        P['g2m_gnn'], grid_h, mesh_h, [g2m_e], g2m_src, g2m_dst,
        N_MESH, 1, 400, same_type=False, need_src_update=True,
        need_e_out=False)

    mesh_h2, _, e_carry = _gnn_step(
        P['m2m_gnns'][0], mesh_h, mesh_h, [m2m_e], m2m_src, m2m_dst,
        N_MESH, 1, 400, same_type=True, need_src_update=True,
        need_e_out=True)
    mesh_h = mesh_h2
    e_parts = list(e_carry[0]) + [e_carry[1]]
    mesh_h2, _, _ = _gnn_step(
        P['m2m_gnns'][1], mesh_h, mesh_h, e_parts, m2m_src, m2m_dst,
        N_MESH, 1, 400, same_type=True, need_src_update=True,
        need_e_out=False)
    mesh_h = mesh_h2

    _, grid_h, _ = _gnn_step(
        P['m2g_gnn'], mesh_h, grid_h, [m2g_e], m2g_src, m2g_dst,
        N_GRID, 4, 400, same_type=False, need_src_update=False,
        need_e_out=False)

    out = _mlp([[grid_h]], P['m2g_out'], 400, use_ln=False)
    return out[:, None, :]
